# baseline (speedup 1.0000x reference)
import functools

import jax
import jax.numpy as jnp
from jax import lax
from jax.experimental import pallas as pl
from jax.experimental.pallas import tpu as pltpu
from jax.experimental.pallas import tpu_sc as plsc

N_GRAPH = 64
N_NODE = 128
N_FEAT = 9
N_ROW = 11
HIDDEN = 768
HHID = HIDDEN // 2
BLANES = 32
NC = 2
NS = 16
GPT = N_GRAPH // NS
C = 8
NCHUNK = N_NODE // C
NT = GPT * NCHUNK
ROWS_PC = C * N_ROW
IDX_PT = GPT * N_NODE * N_ROW
NBUF = 2

N_ATOM = 4609
N_DEG = 512
IN_OFF = 4616
OUT_OFF = IN_OFF + N_DEG + 8
N_TAB = OUT_OFF + N_DEG
ATOM_RPT = 288
DEG_RPT = N_DEG // NS


def _build_kernel():
    mesh = plsc.VectorSubcoreMesh(core_axis_name="c", subcore_axis_name="s")

    @functools.partial(
        pl.kernel,
        mesh=mesh,
        compiler_params=pltpu.CompilerParams(use_tc_tiling_on_sc=False),
        out_type=jax.ShapeDtypeStruct((N_GRAPH, N_NODE, HIDDEN),
                                      jnp.bfloat16),
        scratch_types=[
            pltpu.VMEM((IDX_PT,), jnp.int32),
            pltpu.VMEM((NBUF, ROWS_PC, HHID), jnp.bfloat16),
            pltpu.VMEM((NBUF, C, HHID), jnp.bfloat16),
            pltpu.VMEM_SHARED((N_TAB, HHID), jnp.bfloat16),
            pltpu.SemaphoreType.DMA,
            pltpu.SemaphoreType.DMA,
            pltpu.SemaphoreType.DMA,
            pltpu.SemaphoreType.DMA,
            pltpu.SemaphoreType.DMA,
        ],
    )
    def k(idx_hbm, atom_hbm, ind_hbm, outd_hbm, out_hbm,
          idxv, gbuf, rbuf, spt,
          sem_p, sem_g0, sem_g1, sem_o0, sem_o1):
        cid = lax.axis_index("c")
        sid = lax.axis_index("s")
        sem_g = (sem_g0, sem_g1)
        sem_o = (sem_o0, sem_o1)
        col0 = cid * HHID

        ar0 = sid * ATOM_RPT
        dr0 = sid * DEG_RPT
        pltpu.async_copy(atom_hbm.at[pl.ds(ar0, ATOM_RPT), pl.ds(col0, HHID)],
                         spt.at[pl.ds(ar0, ATOM_RPT)], sem_p)
        pltpu.async_copy(ind_hbm.at[pl.ds(dr0, DEG_RPT), pl.ds(col0, HHID)],
                         spt.at[pl.ds(IN_OFF + dr0, DEG_RPT)], sem_p)
        pltpu.async_copy(outd_hbm.at[pl.ds(dr0, DEG_RPT), pl.ds(col0, HHID)],
                         spt.at[pl.ds(OUT_OFF + dr0, DEG_RPT)], sem_p)

        @pl.when(sid == 0)
        def _():
            pltpu.async_copy(
                atom_hbm.at[pl.ds(N_ATOM - 1, 1), pl.ds(col0, HHID)],
                spt.at[pl.ds(N_ATOM - 1, 1)], sem_p)

        pltpu.async_copy(idx_hbm.at[pl.ds(sid * IDX_PT, IDX_PT)], idxv, sem_p)

        pltpu.make_async_copy(
            atom_hbm.at[pl.ds(0, ATOM_RPT), pl.ds(0, HHID)],
            spt.at[pl.ds(0, ATOM_RPT)], sem_p).wait()
        pltpu.make_async_copy(
            ind_hbm.at[pl.ds(0, DEG_RPT), pl.ds(0, HHID)],
            spt.at[pl.ds(IN_OFF, DEG_RPT)], sem_p).wait()
        pltpu.make_async_copy(
            outd_hbm.at[pl.ds(0, DEG_RPT), pl.ds(0, HHID)],
            spt.at[pl.ds(OUT_OFF, DEG_RPT)], sem_p).wait()
        pltpu.make_async_copy(idx_hbm.at[pl.ds(0, IDX_PT)], idxv, sem_p).wait()

        @pl.when(sid == 0)
        def _():
            pltpu.make_async_copy(
                atom_hbm.at[pl.ds(0, 1), pl.ds(0, HHID)],
                spt.at[pl.ds(0, 1)], sem_p).wait()

        plsc.subcore_barrier()

        def fire_gather(t, b):
            pltpu.async_copy(
                spt.at[idxv.at[pl.ds(t * ROWS_PC, ROWS_PC)]],
                gbuf.at[b], sem_g[b])

        def do_chunk(t, b):
            pltpu.make_async_copy(
                spt.at[idxv.at[pl.ds(0, ROWS_PC)]],
                gbuf.at[b], sem_g[b]).wait()

            @pl.when(t + NBUF - 1 < NT)
            def _():
                fire_gather(t + NBUF - 1, (b + NBUF - 1) % NBUF)

            @pl.when(t >= NBUF)
            def _():
                pltpu.make_async_copy(
                    rbuf.at[b],
                    out_hbm.at[0, pl.ds(0, C), pl.ds(0, HHID)],
                    sem_o[b]).wait()

            @pl.loop(0, C)
            def _node(i):
                @pl.loop(0, HHID // BLANES)
                def _col(j):
                    col = j * BLANES
                    acc = gbuf[b, i * N_ROW, pl.ds(col, BLANES)]
                    for f in range(1, N_ROW):
                        acc = acc + gbuf[b, i * N_ROW + f, pl.ds(col, BLANES)]
                    rbuf[b, i, pl.ds(col, BLANES)] = acc

            g = sid * GPT + lax.div(t, NCHUNK)
            node0 = lax.rem(t, NCHUNK) * C
            pltpu.async_copy(
                rbuf.at[b],
                out_hbm.at[g, pl.ds(node0, C), pl.ds(col0, HHID)],
                sem_o[b])

        for b in range(NBUF - 1):
            fire_gather(b, b)

        @pl.loop(0, NT, step=NBUF)
        def _quad(t0):
            for b in range(NBUF):
                do_chunk(t0 + b, b)

        for b in range(NBUF):
            pltpu.make_async_copy(
                rbuf.at[b], out_hbm.at[0, pl.ds(0, C), pl.ds(0, HHID)],
                sem_o[b]).wait()

    return k


_KERNEL = _build_kernel()


def kernel(x, in_degree, out_degree, atom_table, in_deg_table, out_deg_table,
           graph_token):
    idx = jnp.concatenate(
        [x.astype(jnp.int32),
         in_degree.astype(jnp.int32)[..., None] + IN_OFF,
         out_degree.astype(jnp.int32)[..., None] + OUT_OFF], axis=-1)
    nodes = _KERNEL(idx.reshape(-1),
                    atom_table.astype(jnp.bfloat16),
                    in_deg_table.astype(jnp.bfloat16),
                    out_deg_table.astype(jnp.bfloat16))
    tok = jnp.broadcast_to(graph_token[None, :, :], (N_GRAPH, 1, HIDDEN))
    return jnp.concatenate([tok, nodes.astype(jnp.float32)], axis=1)


# speedup vs baseline: 1.1510x; 1.1510x over previous
import functools

import jax
import jax.numpy as jnp
from jax import lax
from jax.experimental import pallas as pl
from jax.experimental.pallas import tpu as pltpu
from jax.experimental.pallas import tpu_sc as plsc

N_GRAPH = 64
N_NODE = 128
N_FEAT = 9
HIDDEN = 768
HHID = HIDDEN // 2
BLANES = 32
NC = 2
NS = 16
GPT = N_GRAPH // NS
C = 8
NCHUNK = N_NODE // C
NT = GPT * NCHUNK
AROWS_PC = C * N_FEAT
XIDX_PT = GPT * N_NODE * N_FEAT
DIDX_PT = GPT * N_NODE
NBUF = 2

N_ATOM = 4609
N_DEG = 512
ATOM_RPT = 288
DEG_RPT = N_DEG // NS


def _build_kernel():
    mesh = plsc.VectorSubcoreMesh(core_axis_name="c", subcore_axis_name="s")

    @functools.partial(
        pl.kernel,
        mesh=mesh,
        compiler_params=pltpu.CompilerParams(use_tc_tiling_on_sc=False),
        out_type=jax.ShapeDtypeStruct((N_GRAPH, N_NODE + 1, HIDDEN),
                                      jnp.bfloat16),
        scratch_types=[
            pltpu.VMEM((XIDX_PT,), jnp.int32),
            pltpu.VMEM((DIDX_PT,), jnp.int32),
            pltpu.VMEM((DIDX_PT,), jnp.int32),
            pltpu.VMEM((NBUF, AROWS_PC, HHID), jnp.bfloat16),
            pltpu.VMEM((NBUF, C, HHID), jnp.bfloat16),
            pltpu.VMEM((NBUF, C, HHID), jnp.bfloat16),
            pltpu.VMEM((NBUF, C, HHID), jnp.bfloat16),
            pltpu.VMEM((1, HHID), jnp.bfloat16),
            pltpu.VMEM_SHARED((N_ATOM, HHID), jnp.bfloat16),
            pltpu.VMEM_SHARED((N_DEG, HHID), jnp.bfloat16),
            pltpu.VMEM_SHARED((N_DEG, HHID), jnp.bfloat16),
            pltpu.SemaphoreType.DMA,
            pltpu.SemaphoreType.DMA,
            pltpu.SemaphoreType.DMA,
            pltpu.SemaphoreType.DMA,
            pltpu.SemaphoreType.DMA,
        ],
    )
    def k(x_hbm, ind_hbm, outd_hbm, atom_hbm, int_hbm, outt_hbm, tok_hbm,
          out_hbm,
          xidxv, iidxv, oidxv, abuf, ibuf, obuf, rbuf, tokv,
          spa, spi, spo,
          sem_p, sem_g0, sem_g1, sem_o0, sem_o1):
        cid = lax.axis_index("c")
        sid = lax.axis_index("s")
        sem_g = (sem_g0, sem_g1)
        sem_o = (sem_o0, sem_o1)
        col0 = cid * HHID

        ar0 = sid * ATOM_RPT
        dr0 = sid * DEG_RPT
        pltpu.async_copy(atom_hbm.at[pl.ds(ar0, ATOM_RPT), pl.ds(col0, HHID)],
                         spa.at[pl.ds(ar0, ATOM_RPT)], sem_p)
        pltpu.async_copy(int_hbm.at[pl.ds(dr0, DEG_RPT), pl.ds(col0, HHID)],
                         spi.at[pl.ds(dr0, DEG_RPT)], sem_p)
        pltpu.async_copy(outt_hbm.at[pl.ds(dr0, DEG_RPT), pl.ds(col0, HHID)],
                         spo.at[pl.ds(dr0, DEG_RPT)], sem_p)

        @pl.when(sid == 0)
        def _():
            pltpu.async_copy(
                atom_hbm.at[pl.ds(N_ATOM - 1, 1), pl.ds(col0, HHID)],
                spa.at[pl.ds(N_ATOM - 1, 1)], sem_p)

        pltpu.async_copy(x_hbm.at[pl.ds(sid * XIDX_PT, XIDX_PT)], xidxv,
                         sem_p)
        pltpu.async_copy(ind_hbm.at[pl.ds(sid * DIDX_PT, DIDX_PT)], iidxv,
                         sem_p)
        pltpu.async_copy(outd_hbm.at[pl.ds(sid * DIDX_PT, DIDX_PT)], oidxv,
                         sem_p)
        pltpu.async_copy(tok_hbm.at[pl.ds(0, 1), pl.ds(col0, HHID)], tokv,
                         sem_p)

        pltpu.make_async_copy(
            atom_hbm.at[pl.ds(0, ATOM_RPT), pl.ds(0, HHID)],
            spa.at[pl.ds(0, ATOM_RPT)], sem_p).wait()
        pltpu.make_async_copy(
            int_hbm.at[pl.ds(0, DEG_RPT), pl.ds(0, HHID)],
            spi.at[pl.ds(0, DEG_RPT)], sem_p).wait()
        pltpu.make_async_copy(
            outt_hbm.at[pl.ds(0, DEG_RPT), pl.ds(0, HHID)],
            spo.at[pl.ds(0, DEG_RPT)], sem_p).wait()
        pltpu.make_async_copy(x_hbm.at[pl.ds(0, XIDX_PT)], xidxv, sem_p).wait()
        pltpu.make_async_copy(ind_hbm.at[pl.ds(0, DIDX_PT)], iidxv,
                              sem_p).wait()
        pltpu.make_async_copy(outd_hbm.at[pl.ds(0, DIDX_PT)], oidxv,
                              sem_p).wait()
        pltpu.make_async_copy(tok_hbm.at[pl.ds(0, 1), pl.ds(0, HHID)], tokv,
                              sem_p).wait()

        @pl.when(sid == 0)
        def _():
            pltpu.make_async_copy(
                atom_hbm.at[pl.ds(0, 1), pl.ds(0, HHID)],
                spa.at[pl.ds(0, 1)], sem_p).wait()

        plsc.subcore_barrier()

        for gl in range(GPT):
            g = sid * GPT + gl
            pltpu.async_copy(tokv,
                             out_hbm.at[g, pl.ds(0, 1), pl.ds(col0, HHID)],
                             sem_p)

        def fire_gather(t, b):
            pltpu.async_copy(
                spa.at[xidxv.at[pl.ds(t * AROWS_PC, AROWS_PC)]],
                abuf.at[b], sem_g[b])
            pltpu.async_copy(
                spi.at[iidxv.at[pl.ds(t * C, C)]], ibuf.at[b], sem_g[b])
            pltpu.async_copy(
                spo.at[oidxv.at[pl.ds(t * C, C)]], obuf.at[b], sem_g[b])

        def do_chunk(t, b):
            pltpu.make_async_copy(
                spa.at[xidxv.at[pl.ds(0, AROWS_PC)]],
                abuf.at[b], sem_g[b]).wait()
            pltpu.make_async_copy(
                spi.at[iidxv.at[pl.ds(0, C)]], ibuf.at[b], sem_g[b]).wait()
            pltpu.make_async_copy(
                spo.at[oidxv.at[pl.ds(0, C)]], obuf.at[b], sem_g[b]).wait()

            @pl.when(t + NBUF - 1 < NT)
            def _():
                fire_gather(t + NBUF - 1, (b + NBUF - 1) % NBUF)

            @pl.when(t >= NBUF)
            def _():
                pltpu.make_async_copy(
                    rbuf.at[b],
                    out_hbm.at[0, pl.ds(0, C), pl.ds(0, HHID)],
                    sem_o[b]).wait()

            @pl.loop(0, C)
            def _node(i):
                @pl.loop(0, HHID // BLANES)
                def _col(j):
                    col = j * BLANES
                    acc = abuf[b, i * N_FEAT, pl.ds(col, BLANES)]
                    for f in range(1, N_FEAT):
                        acc = acc + abuf[b, i * N_FEAT + f,
                                         pl.ds(col, BLANES)]
                    acc = acc + ibuf[b, i, pl.ds(col, BLANES)]
                    acc = acc + obuf[b, i, pl.ds(col, BLANES)]
                    rbuf[b, i, pl.ds(col, BLANES)] = acc

            g = sid * GPT + lax.div(t, NCHUNK)
            node0 = lax.rem(t, NCHUNK) * C
            pltpu.async_copy(
                rbuf.at[b],
                out_hbm.at[g, pl.ds(1 + node0, C), pl.ds(col0, HHID)],
                sem_o[b])

        for b in range(NBUF - 1):
            fire_gather(b, b)

        @pl.loop(0, NT, step=NBUF)
        def _ring(t0):
            for b in range(NBUF):
                do_chunk(t0 + b, b)

        for b in range(NBUF):
            pltpu.make_async_copy(
                rbuf.at[b], out_hbm.at[0, pl.ds(0, C), pl.ds(0, HHID)],
                sem_o[b]).wait()
        for _ in range(GPT):
            pltpu.make_async_copy(
                tokv, out_hbm.at[0, pl.ds(0, 1), pl.ds(0, HHID)],
                sem_p).wait()

    return k


_KERNEL = _build_kernel()


def kernel(x, in_degree, out_degree, atom_table, in_deg_table, out_deg_table,
           graph_token):
    out = _KERNEL(x.reshape(-1).astype(jnp.int32),
                  in_degree.reshape(-1).astype(jnp.int32),
                  out_degree.reshape(-1).astype(jnp.int32),
                  atom_table.astype(jnp.bfloat16),
                  in_deg_table.astype(jnp.bfloat16),
                  out_deg_table.astype(jnp.bfloat16),
                  graph_token.astype(jnp.bfloat16))
    return out.astype(jnp.float32)
